# Initial kernel scaffold; baseline (speedup 1.0000x reference)
import jax
import jax.numpy as jnp
from jax.experimental import pallas as pl


def kernel(xyzs, features, gamma, beta, W1, b1, W2, b2, W3, b3, W4, b4):
    raise NotImplementedError("write your pallas kernel here")


# speedup vs baseline: 18.4748x; 18.4748x over previous
import jax
import jax.numpy as jnp
from jax import lax
from jax.experimental import pallas as pl


def _body(feat_ref, w_ref, xyzs_ref, gamma_ref, beta_ref, W1_ref, b1_ref,
          W2_ref, b2_ref, xyzs_out_ref, feats_out_ref, idx_out_ref):
    C, N = feat_ref.shape[1], feat_ref.shape[2]
    K = xyzs_out_ref.shape[1]

    x = feat_ref[0]
    mu = jnp.mean(x, axis=0, keepdims=True)
    d = x - mu
    var = jnp.mean(d * d, axis=0, keepdims=True)
    f = d / jnp.sqrt(var + 1e-6) * gamma_ref[...] + beta_ref[...]
    cdim = (((0,), (0,)), ((), ()))
    h = jax.nn.relu(lax.dot_general(W1_ref[...], f, cdim) + b1_ref[...])
    nf = lax.dot_general(W2_ref[...], h, cdim) + b2_ref[...]

    w_row = w_ref[0]
    w_col = jnp.transpose(w_row, (1, 0))
    CH = 256
    chunks = []
    for ci in range(N // CH):
        wi = w_col[ci * CH:(ci + 1) * CH, :]
        gt = w_row > wi
        eq = w_row == wi
        col = lax.broadcasted_iota(jnp.int32, (CH, N), 1)
        row = lax.broadcasted_iota(jnp.int32, (CH, N), 0) + ci * CH
        beats = jnp.where(gt | (eq & (col < row)), 1.0, 0.0)
        chunks.append(jnp.sum(beats, axis=1, keepdims=True))
    rank = jnp.concatenate(chunks, axis=0)
    rank_row = jnp.transpose(rank, (1, 0))

    iota_col = lax.broadcasted_iota(jnp.int32, (N, 1), 0).astype(jnp.float32)
    CK = 256
    for kc in range(K // CK):
        kio = (lax.broadcasted_iota(jnp.int32, (CK, N), 0) + kc * CK
               ).astype(jnp.float32)
        P = jnp.where(rank_row == kio, 1.0, 0.0)
        xyzs_out_ref[0, pl.ds(kc * CK, CK), :] = jnp.dot(P, xyzs_ref[0])
        feats_out_ref[0, :, pl.ds(kc * CK, CK)] = lax.dot_general(
            nf, P, (((1,), (1,)), ((), ())))
        idxf = jnp.dot(P, iota_col, precision=lax.Precision.HIGHEST)
        idx_out_ref[0, :, pl.ds(kc * CK, CK)] = (
            jnp.transpose(idxf, (1, 0)).astype(jnp.int32))


def kernel(xyzs, features, gamma, beta, W1, b1, W2, b2, W3, b3, W4, b4):
    B, C, N = features.shape
    OUTC = W2.shape[1]
    K = 1024

    f0 = jnp.transpose(features, (0, 2, 1))
    mu = jnp.mean(f0, axis=-1, keepdims=True)
    var = jnp.var(f0, axis=-1, keepdims=True)
    fl = (f0 - mu) / jnp.sqrt(var + 1e-6) * gamma + beta
    g = jax.nn.relu(fl @ W3 + b3)
    w = g @ W4 + b4
    w_in = jnp.transpose(w, (0, 2, 1))

    full = lambda s: pl.BlockSpec(s, lambda b: (0,) * len(s))
    perb = lambda s: pl.BlockSpec(s, lambda b: (b,) + (0,) * (len(s) - 1))

    xyzs_out, feats_out, idx3 = pl.pallas_call(
        _body,
        grid=(B,),
        in_specs=[
            perb((1, C, N)),
            perb((1, 1, N)),
            perb((1, N, 3)),
            full((C, 1)), full((C, 1)),
            full((C, C)), full((C, 1)),
            full((C, OUTC)), full((OUTC, 1)),
        ],
        out_specs=[
            perb((1, K, 3)),
            perb((1, OUTC, K)),
            perb((1, 1, K)),
        ],
        out_shape=[
            jax.ShapeDtypeStruct((B, K, 3), jnp.float32),
            jax.ShapeDtypeStruct((B, OUTC, K), jnp.float32),
            jax.ShapeDtypeStruct((B, 1, K), jnp.int32),
        ],
    )(features, w_in, xyzs, gamma.reshape(C, 1), beta.reshape(C, 1),
      W1, b1.reshape(C, 1), W2, b2.reshape(OUTC, 1))

    return xyzs_out, feats_out, idx3.reshape(B, K)
